# baseline (speedup 1.0000x reference)
import functools

import jax
import jax.numpy as jnp
from jax import lax
from jax.experimental import pallas as pl
from jax.experimental.pallas import tpu as pltpu
from jax.experimental.pallas import tpu_sc as plsc

_LANES = 16
_UNROLL = 14
_NBLK = 2


def _make_sc_kernel(n, chunk, num_cores, n_species):
    mesh = plsc.VectorSubcoreMesh(
        core_axis_name="c", subcore_axis_name="s", num_cores=num_cores)
    blk = chunk // _NBLK

    @functools.partial(
        pl.kernel,
        mesh=mesh,
        out_type=jax.ShapeDtypeStruct((n,), jnp.float32),
        scratch_types=[
            pltpu.VMEM((chunk,), jnp.float32),
            pltpu.VMEM((chunk,), jnp.int32),
            pltpu.VMEM((_LANES,), jnp.float32),
            pltpu.VMEM((_LANES,), jnp.float32),
            pltpu.VMEM((chunk,), jnp.float32),
            pltpu.SemaphoreType.DMA,
            pltpu.SemaphoreType.DMA,
            pltpu.SemaphoreType.DMA,
            pltpu.SemaphoreType.DMA,
        ],
    )
    def k(x_hbm, t_hbm, scale_hbm, shift_hbm, out_hbm, x_v, t_v, sc_v, sh_v, o_v,
          sem_tab, sem0, sem1, sem_out):
        wid = lax.axis_index("s") * num_cores + lax.axis_index("c")
        base = jnp.minimum(wid * chunk, n - chunk)
        ct = pltpu.async_copy(scale_hbm, sc_v.at[pl.ds(0, n_species)], sem_tab)
        ct2 = pltpu.async_copy(shift_hbm, sh_v.at[pl.ds(0, n_species)], sem_tab)
        sems = (sem0, sem1)
        incs = []
        for b in range(_NBLK):
            s = pl.ds(b * blk, blk)
            incs.append((
                pltpu.async_copy(x_hbm.at[pl.ds(base + b * blk, blk)],
                                 x_v.at[s], sems[b]),
                pltpu.async_copy(t_hbm.at[pl.ds(base + b * blk, blk)],
                                 t_v.at[s], sems[b]),
            ))
        ct.wait()
        ct2.wait()
        sc_vec = sc_v[...]
        sh_vec = sh_v[...]

        outs = []
        for b in range(_NBLK):
            incs[b][0].wait()
            incs[b][1].wait()

            def body(i, carry, _b=b):
                for u in range(_UNROLL):
                    sl = pl.ds(_b * blk + (i * _UNROLL + u) * _LANES, _LANES)
                    t = t_v[sl]
                    s = sc_vec.at[t].get(mode="promise_in_bounds")
                    sh = sh_vec.at[t].get(mode="promise_in_bounds")
                    o_v[sl] = x_v[sl] * s + sh
                return carry

            lax.fori_loop(0, blk // (_LANES * _UNROLL), body, 0)
            outs.append(pltpu.async_copy(
                o_v.at[pl.ds(b * blk, blk)],
                out_hbm.at[pl.ds(base + b * blk, blk)], sem_out))
        for c in outs:
            c.wait()

    return k


def kernel(scaled_atomic_energy, atom_type, shift, scale):
    n = scaled_atomic_energy.shape[0]
    n_species = scale.shape[0]
    info = plsc.get_sparse_core_info()
    num_cores = info.num_cores
    num_workers = num_cores * info.num_subcores
    grain = _LANES * _UNROLL * _NBLK
    chunk = -(-n // num_workers)
    chunk = -(-chunk // grain) * grain

    x = scaled_atomic_energy.reshape(-1)
    out = _make_sc_kernel(n, chunk, num_cores, n_species)(
        x, atom_type, scale, shift)
    return out.reshape(n, 1)


# speedup vs baseline: 1.0082x; 1.0082x over previous
import functools

import jax
import jax.numpy as jnp
from jax import lax
from jax.experimental import pallas as pl
from jax.experimental.pallas import tpu as pltpu
from jax.experimental.pallas import tpu_sc as plsc

_LANES = 16
_UNROLL = 14


def _make_sc_kernel(n, chunk, num_cores, n_species):
    mesh = plsc.VectorSubcoreMesh(
        core_axis_name="c", subcore_axis_name="s", num_cores=num_cores)
    rows = chunk // _LANES

    @functools.partial(
        pl.kernel,
        mesh=mesh,
        out_type=jax.ShapeDtypeStruct((n,), jnp.float32),
        scratch_types=[
            pltpu.VMEM((chunk,), jnp.float32),
            pltpu.VMEM((chunk,), jnp.int32),
            pltpu.VMEM((_LANES,), jnp.float32),
            pltpu.VMEM((_LANES,), jnp.float32),
            pltpu.VMEM((chunk,), jnp.float32),
            pltpu.SemaphoreType.DMA,
        ],
    )
    def k(x_hbm, t_hbm, scale_hbm, shift_hbm, out_hbm, x_v, t_v, sc_v, sh_v, o_v,
          sem):
        wid = lax.axis_index("s") * num_cores + lax.axis_index("c")
        base = jnp.minimum(wid * chunk, n - chunk)
        c1 = pltpu.async_copy(x_hbm.at[pl.ds(base, chunk)], x_v, sem)
        c2 = pltpu.async_copy(t_hbm.at[pl.ds(base, chunk)], t_v, sem)
        c3 = pltpu.async_copy(scale_hbm, sc_v.at[pl.ds(0, n_species)], sem)
        c4 = pltpu.async_copy(shift_hbm, sh_v.at[pl.ds(0, n_species)], sem)
        c1.wait()
        c2.wait()
        c3.wait()
        c4.wait()

        sc_vec = sc_v[...]
        sh_vec = sh_v[...]

        def body(i, carry):
            for u in range(_UNROLL):
                r = i * _UNROLL + u
                sl = pl.ds(r * _LANES, _LANES)
                t = t_v[sl]
                s = sc_vec.at[t].get(mode="promise_in_bounds")
                b = sh_vec.at[t].get(mode="promise_in_bounds")
                o_v[sl] = x_v[sl] * s + b
            return carry

        lax.fori_loop(0, rows // _UNROLL, body, 0)
        pltpu.sync_copy(o_v, out_hbm.at[pl.ds(base, chunk)])

    return k


def kernel(scaled_atomic_energy, atom_type, shift, scale):
    n = scaled_atomic_energy.shape[0]
    n_species = scale.shape[0]
    info = plsc.get_sparse_core_info()
    num_cores = info.num_cores
    num_workers = num_cores * info.num_subcores
    grain = _LANES * _UNROLL
    chunk = -(-n // num_workers)
    chunk = -(-chunk // grain) * grain

    x = scaled_atomic_energy.reshape(-1)
    out = _make_sc_kernel(n, chunk, num_cores, n_species)(
        x, atom_type, scale, shift)
    return out.reshape(n, 1)


# speedup vs baseline: 1.0103x; 1.0021x over previous
import functools

import jax
import jax.numpy as jnp
from jax import lax
from jax.experimental import pallas as pl
from jax.experimental.pallas import tpu as pltpu
from jax.experimental.pallas import tpu_sc as plsc

_LANES = 16
_UNROLL = 14


def _make_sc_kernel(n, chunk, num_cores, n_species):
    mesh = plsc.VectorSubcoreMesh(
        core_axis_name="c", subcore_axis_name="s", num_cores=num_cores)
    rows = chunk // _LANES

    @functools.partial(
        pl.kernel,
        mesh=mesh,
        out_type=jax.ShapeDtypeStruct((n,), jnp.float32),
        scratch_types=[
            pltpu.VMEM((chunk,), jnp.float32),
            pltpu.VMEM((chunk,), jnp.int32),
            pltpu.VMEM((_LANES,), jnp.float32),
            pltpu.VMEM((_LANES,), jnp.float32),
            pltpu.VMEM((chunk,), jnp.float32),
            pltpu.SemaphoreType.DMA,
        ],
    )
    def k(x_hbm, t_hbm, scale_hbm, shift_hbm, out_hbm, x_v, t_v, sc_v, sh_v, o_v,
          sem):
        wid = lax.axis_index("s") * num_cores + lax.axis_index("c")
        base = jnp.minimum(wid * chunk, n - chunk)
        c1 = pltpu.async_copy(x_hbm.at[pl.ds(base, chunk)], x_v, sem)
        c2 = pltpu.async_copy(t_hbm.at[pl.ds(base, chunk)], t_v, sem)
        c3 = pltpu.async_copy(scale_hbm, sc_v.at[pl.ds(0, n_species)], sem)
        c4 = pltpu.async_copy(shift_hbm, sh_v.at[pl.ds(0, n_species)], sem)
        c1.wait()
        c2.wait()
        c3.wait()
        c4.wait()

        sc_vec = sc_v[...]
        sh_vec = sh_v[...]

        def body(i, carry):
            for u in range(_UNROLL):
                sl = pl.ds((i * _UNROLL + u) * _LANES, _LANES)
                t = t_v[sl]
                s = sc_vec.at[t].get(mode="promise_in_bounds")
                b = sh_vec.at[t].get(mode="promise_in_bounds")
                o_v[sl] = x_v[sl] * s + b
            return carry

        lax.fori_loop(0, rows // _UNROLL, body, 0)
        pltpu.sync_copy(o_v, out_hbm.at[pl.ds(base, chunk)])

    return k


def kernel(scaled_atomic_energy, atom_type, shift, scale):
    n = scaled_atomic_energy.shape[0]
    n_species = scale.shape[0]
    info = plsc.get_sparse_core_info()
    num_cores = info.num_cores
    num_workers = num_cores * info.num_subcores
    grain = _LANES * _UNROLL
    chunk = -(-n // num_workers)
    chunk = -(-chunk // grain) * grain

    x = scaled_atomic_energy.reshape(-1)
    out = _make_sc_kernel(n, chunk, num_cores, n_species)(
        x, atom_type, scale, shift)
    return out.reshape(n, 1)
